# baseline (speedup 1.0000x reference)
import jax
import jax.numpy as jnp
from jax import lax
from jax.experimental import pallas as pl
from jax.experimental.pallas import tpu as pltpu
from jax.experimental.pallas import tpu_sc as plsc

_NC, _NS = 2, 16
_NW = _NC * _NS
_R = 16
_L = 16
_E = 512
_BS = 512


def _sc_body(x_hbm, pe_hbm, out_hbm, pebuf, xbuf, lsem, ssem):
    s_full = pe_hbm.shape[0]
    h = pe_hbm.shape[1]
    n_batch = x_hbm.shape[0] // s_full
    parts = _NW // n_batch
    rows_per_part = _E // parts
    n_chunks = rows_per_part // _R
    seq_lo = s_full - _E

    wid = lax.axis_index("s") * _NC + lax.axis_index("c")
    b = wid // parts
    part = lax.rem(wid, parts)
    pe_base = seq_lo + part * rows_per_part
    x_base = b * s_full + pe_base
    out_base = b * _E + part * rows_per_part

    def start_loads(c, pb):
        r0 = c * _R
        pltpu.make_async_copy(
            pe_hbm.at[pl.ds(pe_base + r0, _R), :], pebuf.at[pb], lsem
        ).start()
        pltpu.make_async_copy(
            x_hbm.at[pl.ds(x_base + r0, _R), :], xbuf.at[pb], lsem
        ).start()

    def wait_loads(pb):
        pltpu.make_async_copy(pe_hbm.at[pl.ds(0, _R), :], pebuf.at[pb], lsem).wait()
        pltpu.make_async_copy(x_hbm.at[pl.ds(0, _R), :], xbuf.at[pb], lsem).wait()

    def start_store(c, pb):
        pltpu.make_async_copy(
            xbuf.at[pb], out_hbm.at[pl.ds(out_base + c * _R, _R), :], ssem
        ).start()

    def drain_one_store(pb):
        pltpu.make_async_copy(
            xbuf.at[pb], out_hbm.at[pl.ds(0, _R), :], ssem
        ).wait()

    start_loads(0, 0)

    def step(c, _):
        pb = lax.rem(c, 2)
        wait_loads(pb)

        @pl.when(c + 1 < n_chunks)
        def _():
            @pl.when(c >= 1)
            def _():
                drain_one_store(1 - pb)

            start_loads(c + 1, 1 - pb)

        for r in range(_R):
            @plsc.parallel_loop(0, h, _L, unroll=8)
            def _(j, r=r):
                xbuf[pb, r, pl.ds(j, _L)] = (
                    xbuf[pb, r, pl.ds(j, _L)] + pebuf[pb, r, pl.ds(j, _L)]
                )

        start_store(c, pb)
        return 0

    lax.fori_loop(0, n_chunks, step, 0)

    for _i in range(2):
        drain_one_store(0)


def _tc_body(x_ref, pe_ref, out_ref):
    out_ref[0] = x_ref[0] + pe_ref[...]


def kernel(x, pe_table):
    B, S, H = x.shape
    s_tc = S - _E
    x2d = x.reshape(B * S, H)

    mesh = plsc.VectorSubcoreMesh(
        core_axis_name="c", subcore_axis_name="s", num_cores=_NC, num_subcores=_NS
    )
    sc_out = pl.kernel(
        _sc_body,
        out_type=jax.ShapeDtypeStruct((B * _E, H), x.dtype),
        mesh=mesh,
        scratch_types=[
            pltpu.VMEM((2, _R, H), x.dtype),
            pltpu.VMEM((2, _R, H), x.dtype),
            pltpu.SemaphoreType.DMA,
            pltpu.SemaphoreType.DMA,
        ],
    )(x2d, pe_table[:S])

    tc_out = pl.pallas_call(
        _tc_body,
        grid=(s_tc // _BS, B),
        in_specs=[
            pl.BlockSpec((1, _BS, H), lambda s, b: (b, s, 0)),
            pl.BlockSpec((_BS, H), lambda s, b: (s, 0)),
        ],
        out_specs=pl.BlockSpec((1, _BS, H), lambda s, b: (b, s, 0)),
        out_shape=jax.ShapeDtypeStruct((B, s_tc, H), x.dtype),
    )(x, pe_table)

    return jnp.concatenate([tc_out, sc_out.reshape(B, _E, H)], axis=1)
